# baseline (speedup 1.0000x reference)
import functools

import jax
import jax.numpy as jnp
from jax import lax
from jax.experimental import pallas as pl
from jax.experimental.pallas import tpu as pltpu
from jax.experimental.pallas import tpu_sc as plsc


CHUNK = 200
NBUF = 4
TBL = 512


def _transpose_pad_body(in_ref, out_ref):
    dim = in_ref.shape[0]
    out_ref[:, :dim] = in_ref[:, :].T


def _transpose_pad(tt, pdim):
    dim, vocab = tt.shape
    return pl.pallas_call(
        _transpose_pad_body,
        grid=(pl.cdiv(vocab, TBL),),
        in_specs=[pl.BlockSpec((dim, TBL), lambda i: (0, i))],
        out_specs=pl.BlockSpec((TBL, pdim), lambda i: (i, 0)),
        out_shape=jax.ShapeDtypeStruct((vocab, pdim), jnp.float32),
    )(tt)


@jax.jit
def kernel(x, table):
    batch, hist = x.shape
    n = batch * hist
    dim = table.shape[1]
    pdim = 128
    flat_idx = x.T.reshape(n).astype(jnp.int32)
    tpad = _transpose_pad(table.T, pdim)

    info = plsc.get_sparse_core_info()
    num_workers = info.num_cores * info.num_subcores
    rows_per_worker = n // num_workers
    assert n % num_workers == 0
    assert rows_per_worker % CHUNK == 0
    n_chunks = rows_per_worker // CHUNK

    mesh = plsc.VectorSubcoreMesh(core_axis_name="c", subcore_axis_name="s")

    @functools.partial(
        pl.kernel,
        mesh=mesh,
        compiler_params=pltpu.CompilerParams(use_tc_tiling_on_sc=True),
        out_type=jax.ShapeDtypeStruct((n, pdim), jnp.float32),
        scratch_types=[
            pltpu.VMEM((rows_per_worker,), jnp.int32),
            pltpu.VMEM((NBUF, CHUNK, pdim), jnp.float32),
            pltpu.SemaphoreType.DMA((NBUF,)),
            pltpu.SemaphoreType.DMA((NBUF,)),
        ],
    )
    def gather_kernel(idx_hbm, table_hbm, out_hbm, idx_v, rows_v, gsem, ssem):
        wid = lax.axis_index("s") * info.num_cores + lax.axis_index("c")
        base_w = wid * rows_per_worker
        pltpu.sync_copy(idx_hbm.at[pl.ds(base_w, rows_per_worker)], idx_v)

        gh = [None] * NBUF
        sh = [None] * NBUF
        for b in range(NBUF):
            gh[b] = pltpu.async_copy(
                table_hbm.at[idx_v.at[pl.ds(b * CHUNK, CHUNK)]],
                rows_v.at[b], gsem.at[b])
        for c in range(n_chunks):
            b = c % NBUF
            gh[b].wait()
            sh[b] = pltpu.async_copy(
                rows_v.at[b], out_hbm.at[pl.ds(base_w + c * CHUNK, CHUNK)],
                ssem.at[b])
            nxt = c + NBUF
            if nxt < n_chunks:
                sh[b].wait()
                gh[b] = pltpu.async_copy(
                    table_hbm.at[idx_v.at[pl.ds(nxt * CHUNK, CHUNK)]],
                    rows_v.at[b], gsem.at[b])
        for c in range(max(0, n_chunks - NBUF), n_chunks):
            sh[c % NBUF].wait()

    out = gather_kernel(flat_idx, tpad)
    return out[:, :dim].reshape(hist, batch, dim).transpose(1, 0, 2)


# speedup vs baseline: 2.8392x; 2.8392x over previous
import functools

import jax
import jax.numpy as jnp
from jax import lax
from jax.experimental import pallas as pl
from jax.experimental.pallas import tpu as pltpu
from jax.experimental.pallas import tpu_sc as plsc


CHUNK = 200
NBUF = 4
TBL = 4096


def _transpose_pad_body(in_ref, out_ref):
    dim = in_ref.shape[0]
    out_ref[:, :dim] = in_ref[:, :].T


def _transpose_pad(tt, pdim):
    dim, vocab = tt.shape
    return pl.pallas_call(
        _transpose_pad_body,
        grid=(pl.cdiv(vocab, TBL),),
        in_specs=[pl.BlockSpec((dim, TBL), lambda i: (0, i))],
        out_specs=pl.BlockSpec((TBL, pdim), lambda i: (i, 0)),
        out_shape=jax.ShapeDtypeStruct((vocab, pdim), jnp.float32),
    )(tt)


@jax.jit
def kernel(x, table):
    batch, hist = x.shape
    n = batch * hist
    dim = table.shape[1]
    pdim = 128
    flat_idx = x.T.reshape(n).astype(jnp.int32)
    tpad = _transpose_pad(table.T, pdim)

    info = plsc.get_sparse_core_info()
    num_workers = info.num_cores * info.num_subcores
    rows_per_worker = n // num_workers
    assert n % num_workers == 0
    assert rows_per_worker % CHUNK == 0
    n_chunks = rows_per_worker // CHUNK

    mesh = plsc.VectorSubcoreMesh(core_axis_name="c", subcore_axis_name="s")

    @functools.partial(
        pl.kernel,
        mesh=mesh,
        compiler_params=pltpu.CompilerParams(use_tc_tiling_on_sc=True),
        out_type=jax.ShapeDtypeStruct((n, pdim), jnp.float32),
        scratch_types=[
            pltpu.VMEM((rows_per_worker,), jnp.int32),
            pltpu.VMEM((NBUF, CHUNK, pdim), jnp.float32),
            pltpu.SemaphoreType.DMA((NBUF,)),
            pltpu.SemaphoreType.DMA((NBUF,)),
        ],
    )
    def gather_kernel(idx_hbm, table_hbm, out_hbm, idx_v, rows_v, gsem, ssem):
        wid = lax.axis_index("s") * info.num_cores + lax.axis_index("c")
        base_w = wid * rows_per_worker
        pltpu.sync_copy(idx_hbm.at[pl.ds(base_w, rows_per_worker)], idx_v)

        gh = [None] * NBUF
        sh = [None] * NBUF
        for b in range(NBUF):
            gh[b] = pltpu.async_copy(
                table_hbm.at[idx_v.at[pl.ds(b * CHUNK, CHUNK)]],
                rows_v.at[b], gsem.at[b])
        for c in range(n_chunks):
            b = c % NBUF
            gh[b].wait()
            sh[b] = pltpu.async_copy(
                rows_v.at[b], out_hbm.at[pl.ds(base_w + c * CHUNK, CHUNK)],
                ssem.at[b])
            nxt = c + NBUF
            if nxt < n_chunks:
                sh[b].wait()
                gh[b] = pltpu.async_copy(
                    table_hbm.at[idx_v.at[pl.ds(nxt * CHUNK, CHUNK)]],
                    rows_v.at[b], gsem.at[b])
        for c in range(max(0, n_chunks - NBUF), n_chunks):
            sh[c % NBUF].wait()

    out = gather_kernel(flat_idx, tpad)
    return out[:, :dim].reshape(hist, batch, dim).transpose(1, 0, 2)


# speedup vs baseline: 3.3692x; 1.1867x over previous
import functools

import jax
import jax.numpy as jnp
from jax import lax
from jax.experimental import pallas as pl
from jax.experimental.pallas import tpu as pltpu
from jax.experimental.pallas import tpu_sc as plsc


CHUNK = 200
NBUF = 4
TBL = 8192


def _transpose_pad_body(in_ref, out_ref):
    dim = in_ref.shape[0]
    out_ref[:, :dim] = in_ref[:, :].T


def _transpose_pad(tt, pdim):
    dim, vocab = tt.shape
    return pl.pallas_call(
        _transpose_pad_body,
        grid=(pl.cdiv(vocab, TBL),),
        in_specs=[pl.BlockSpec((dim, TBL), lambda i: (0, i))],
        out_specs=pl.BlockSpec((TBL, pdim), lambda i: (i, 0)),
        out_shape=jax.ShapeDtypeStruct((vocab, pdim), jnp.float32),
    )(tt)


@jax.jit
def kernel(x, table):
    batch, hist = x.shape
    n = batch * hist
    dim = table.shape[1]
    pdim = 128
    flat_idx = x.T.reshape(n).astype(jnp.int32)
    tpad = _transpose_pad(table.T, pdim)

    info = plsc.get_sparse_core_info()
    num_workers = info.num_cores * info.num_subcores
    rows_per_worker = n // num_workers
    assert n % num_workers == 0
    assert rows_per_worker % CHUNK == 0
    n_chunks = rows_per_worker // CHUNK

    mesh = plsc.VectorSubcoreMesh(core_axis_name="c", subcore_axis_name="s")

    @functools.partial(
        pl.kernel,
        mesh=mesh,
        compiler_params=pltpu.CompilerParams(use_tc_tiling_on_sc=True),
        out_type=jax.ShapeDtypeStruct((n, pdim), jnp.float32),
        scratch_types=[
            pltpu.VMEM((rows_per_worker,), jnp.int32),
            pltpu.VMEM((NBUF, CHUNK, pdim), jnp.float32),
            pltpu.SemaphoreType.DMA((NBUF,)),
            pltpu.SemaphoreType.DMA((NBUF,)),
        ],
    )
    def gather_kernel(idx_hbm, table_hbm, out_hbm, idx_v, rows_v, gsem, ssem):
        wid = lax.axis_index("s") * info.num_cores + lax.axis_index("c")
        base_w = wid * rows_per_worker
        pltpu.sync_copy(idx_hbm.at[pl.ds(base_w, rows_per_worker)], idx_v)

        gh = [None] * NBUF
        sh = [None] * NBUF
        for b in range(NBUF):
            gh[b] = pltpu.async_copy(
                table_hbm.at[idx_v.at[pl.ds(b * CHUNK, CHUNK)]],
                rows_v.at[b], gsem.at[b])
        for c in range(n_chunks):
            b = c % NBUF
            gh[b].wait()
            sh[b] = pltpu.async_copy(
                rows_v.at[b], out_hbm.at[pl.ds(base_w + c * CHUNK, CHUNK)],
                ssem.at[b])
            nxt = c + NBUF
            if nxt < n_chunks:
                sh[b].wait()
                gh[b] = pltpu.async_copy(
                    table_hbm.at[idx_v.at[pl.ds(nxt * CHUNK, CHUNK)]],
                    rows_v.at[b], gsem.at[b])
        for c in range(max(0, n_chunks - NBUF), n_chunks):
            sh[c % NBUF].wait()

    out = gather_kernel(flat_idx, tpad)
    return out[:, :dim].reshape(hist, batch, dim).transpose(1, 0, 2)


# speedup vs baseline: 3.5994x; 1.0683x over previous
import functools

import jax
import jax.numpy as jnp
from jax import lax
from jax.experimental import pallas as pl
from jax.experimental.pallas import tpu as pltpu
from jax.experimental.pallas import tpu_sc as plsc


CHUNK = 200
NBUF = 4
TBL = 32768


def _transpose_pad_body(in_ref, out_ref):
    dim = in_ref.shape[0]
    out_ref[:, :dim] = in_ref[:, :].T


def _transpose_pad(tt, pdim):
    dim, vocab = tt.shape
    return pl.pallas_call(
        _transpose_pad_body,
        grid=(pl.cdiv(vocab, TBL),),
        in_specs=[pl.BlockSpec((dim, TBL), lambda i: (0, i))],
        out_specs=pl.BlockSpec((TBL, pdim), lambda i: (i, 0)),
        out_shape=jax.ShapeDtypeStruct((vocab, pdim), jnp.float32),
    )(tt)


@jax.jit
def kernel(x, table):
    batch, hist = x.shape
    n = batch * hist
    dim = table.shape[1]
    pdim = 128
    flat_idx = x.T.reshape(n).astype(jnp.int32)
    tpad = _transpose_pad(table.T, pdim)

    info = plsc.get_sparse_core_info()
    num_workers = info.num_cores * info.num_subcores
    rows_per_worker = n // num_workers
    assert n % num_workers == 0
    assert rows_per_worker % CHUNK == 0
    n_chunks = rows_per_worker // CHUNK

    mesh = plsc.VectorSubcoreMesh(core_axis_name="c", subcore_axis_name="s")

    @functools.partial(
        pl.kernel,
        mesh=mesh,
        compiler_params=pltpu.CompilerParams(use_tc_tiling_on_sc=True),
        out_type=jax.ShapeDtypeStruct((n, pdim), jnp.float32),
        scratch_types=[
            pltpu.VMEM((rows_per_worker,), jnp.int32),
            pltpu.VMEM((NBUF, CHUNK, pdim), jnp.float32),
            pltpu.SemaphoreType.DMA((NBUF,)),
            pltpu.SemaphoreType.DMA((NBUF,)),
        ],
    )
    def gather_kernel(idx_hbm, table_hbm, out_hbm, idx_v, rows_v, gsem, ssem):
        wid = lax.axis_index("s") * info.num_cores + lax.axis_index("c")
        base_w = wid * rows_per_worker
        pltpu.sync_copy(idx_hbm.at[pl.ds(base_w, rows_per_worker)], idx_v)

        gh = [None] * NBUF
        sh = [None] * NBUF
        for b in range(NBUF):
            gh[b] = pltpu.async_copy(
                table_hbm.at[idx_v.at[pl.ds(b * CHUNK, CHUNK)]],
                rows_v.at[b], gsem.at[b])
        for c in range(n_chunks):
            b = c % NBUF
            gh[b].wait()
            sh[b] = pltpu.async_copy(
                rows_v.at[b], out_hbm.at[pl.ds(base_w + c * CHUNK, CHUNK)],
                ssem.at[b])
            nxt = c + NBUF
            if nxt < n_chunks:
                sh[b].wait()
                gh[b] = pltpu.async_copy(
                    table_hbm.at[idx_v.at[pl.ds(nxt * CHUNK, CHUNK)]],
                    rows_v.at[b], gsem.at[b])
        for c in range(max(0, n_chunks - NBUF), n_chunks):
            sh[c % NBUF].wait()

    out = gather_kernel(flat_idx, tpad)
    return out[:, :dim].reshape(hist, batch, dim).transpose(1, 0, 2)
